# baseline (speedup 1.0000x reference)
import functools

import jax
import jax.numpy as jnp
import numpy as np
from jax import lax
from jax.experimental import pallas as pl
from jax.experimental.pallas import tpu as pltpu
from jax.experimental.pallas import tpu_sc as plsc

_MARGINS = np.array(
    [0.45357266, 1.0, 0.49222963, 0.76696184, 1.0, 0.43823621, 0.60325897,
     0.57481898],
    dtype=np.float32,
)
_M = (2.4 * _MARGINS).astype(np.float32)
_C = 8

_LN_COEFS = (
    0.999999906, -0.4999999891, 0.3333472956, -0.2500141049, 0.1994491456,
    -0.165733744, 0.1505316593, -0.1430870956, 0.08404783038,
)
_LN2 = 0.6931471805599453
_SQRT2 = 1.4142135623730951

_ROWS = 8192
_ROWS_SC = 512
_ROWS_TC = _ROWS - _ROWS_SC
_N_SC = _ROWS_SC * 128
_N_TC = _ROWS_TC * 128


def _tc_body(x_ref, enc_ref, sum_ref, cnt_ref):
    i = pl.program_id(0)

    @pl.when(i == 0)
    def _init():
        sum_ref[0, 0] = jnp.float32(0.0)
        cnt_ref[0, 0] = jnp.float32(0.0)

    Bb = enc_ref.shape[0]
    CH = 64

    acc = jnp.zeros((8, 128), jnp.float32)
    cnt = jnp.zeros((8, 128), jnp.float32)
    for k in range(Bb // CH):
        r = slice(k * CH, (k + 1) * CH)
        enc = enc_ref[r, :].astype(jnp.int32)
        tgt = enc & 7
        mskf = (enc >> 3).astype(jnp.float32)
        xt = jnp.transpose(x_ref[r, :, :], (1, 0, 2))
        S = jnp.zeros((CH, 128), jnp.float32)
        gacc = jnp.zeros((CH, 128), jnp.float32)
        for c in range(_C):
            xc = xt[c]
            sel = tgt == c
            xm = xc - jnp.float32(_M[c])
            S = S + jnp.exp(jnp.where(sel, xm, xc))
            gacc = gacc + jnp.where(sel, xm, 0.0)
        per = mskf * (jnp.log(S) - gacc)
        acc = acc + jnp.sum(per.reshape(CH // 8, 8, 128), axis=0)
        cnt = cnt + jnp.sum(mskf.reshape(CH // 8, 8, 128), axis=0)

    sum_ref[0, 0] += jnp.sum(acc)
    cnt_ref[0, 0] += jnp.sum(cnt)


def _ln16(s):
    b = lax.bitcast_convert_type(s, jnp.int32)
    k = lax.shift_right_arithmetic(b, 23) - 127
    m = lax.bitcast_convert_type(
        (b & jnp.int32(0x007FFFFF)) | jnp.int32(0x3F800000), jnp.float32)
    big = m > jnp.float32(_SQRT2)
    m = jnp.where(big, m * jnp.float32(0.5), m)
    k = jnp.where(big, k + 1, k)
    z = m - jnp.float32(1.0)
    p = jnp.float32(_LN_COEFS[-1])
    for coef in _LN_COEFS[-2::-1]:
        p = p * z + jnp.float32(coef)
    return k.astype(jnp.float32) * jnp.float32(_LN2) + p * z


def _make_sc_kernel():
    info = plsc.get_sparse_core_info()
    nw = info.num_cores * info.num_subcores
    rpt = _ROWS_SC // nw
    npt = rpt * 128
    mesh = plsc.VectorSubcoreMesh(core_axis_name="c", subcore_axis_name="s")

    @functools.partial(
        pl.kernel,
        mesh=mesh,
        out_type=[
            jax.ShapeDtypeStruct((nw, 16), jnp.float32),
            jax.ShapeDtypeStruct((nw, 16), jnp.float32),
        ],
        scratch_types=[
            pltpu.VMEM((rpt, _C, 128), jnp.float32),
            pltpu.VMEM((npt,), jnp.int32),
            pltpu.VMEM((npt,), jnp.int32),
            pltpu.VMEM((16,), jnp.float32),
            pltpu.VMEM((16,), jnp.float32),
        ],
    )
    def sc_loss(x_hbm, tgt_hbm, msk_hbm, sum_hbm, cnt_hbm,
                xbuf, tgtbuf, mskbuf, srow, crow):
        wid = lax.axis_index("s") * info.num_cores + lax.axis_index("c")
        row0 = _ROWS_TC + wid * rpt
        s0 = row0 * 128
        pltpu.sync_copy(x_hbm.at[pl.ds(row0, rpt)], xbuf)
        pltpu.sync_copy(tgt_hbm.at[pl.ds(s0, npt)], tgtbuf)
        pltpu.sync_copy(msk_hbm.at[pl.ds(s0 - _N_TC, npt)], mskbuf)

        def body(j, carry):
            acc, cnt = carry
            r = j // 8
            v = j - r * 8
            tv = tgtbuf[pl.ds(j * 16, 16)]
            mv = mskbuf[pl.ds(j * 16, 16)].astype(jnp.float32)
            s = jnp.zeros((16,), jnp.float32)
            g = jnp.zeros((16,), jnp.float32)
            for c in range(_C):
                xc = xbuf[r, c, pl.ds(v * 16, 16)]
                sel = tv == c
                xm = xc - jnp.float32(_M[c])
                s = s + jnp.exp(jnp.where(sel, xm, xc))
                g = g + jnp.where(sel, xm, jnp.float32(0.0))
            per = (_ln16(s) - g) * mv
            return acc + per, cnt + mv

        acc, cnt = lax.fori_loop(
            0, npt // 16, body,
            (jnp.zeros((16,), jnp.float32), jnp.zeros((16,), jnp.float32)))
        srow[...] = acc
        crow[...] = cnt
        pltpu.sync_copy(srow, sum_hbm.at[wid])
        pltpu.sync_copy(crow, cnt_hbm.at[wid])

    return sc_loss


_SC_LOSS = None


def _sc_loss_fn():
    global _SC_LOSS
    if _SC_LOSS is None:
        _SC_LOSS = _make_sc_kernel()
    return _SC_LOSS


@jax.jit
def kernel(x, target, mask):
    N, C = x.shape
    assert C == _C
    rows = N // 128
    xv = x.reshape(rows, 128, C).transpose(0, 2, 1)

    msk_sc = lax.slice(mask, (_N_TC,), (N,)).astype(jnp.int32)
    sc_s, sc_c = _sc_loss_fn()(xv, target, msk_sc)

    enc = (lax.slice(target, (0,), (_N_TC,))
           | (lax.slice(mask, (0,), (_N_TC,)).astype(jnp.int32) << 3))
    enc = enc.astype(jnp.int8).reshape(_ROWS_TC, 128)

    Bb = 1920
    grid = (_ROWS_TC // Bb,)
    out_shape = [
        jax.ShapeDtypeStruct((1, 1), jnp.float32),
        jax.ShapeDtypeStruct((1, 1), jnp.float32),
    ]
    s, c = pl.pallas_call(
        _tc_body,
        grid=grid,
        in_specs=[
            pl.BlockSpec((Bb, C, 128), lambda i: (i, 0, 0)),
            pl.BlockSpec((Bb, 128), lambda i: (i, 0)),
        ],
        out_specs=[
            pl.BlockSpec(memory_space=pltpu.SMEM),
            pl.BlockSpec(memory_space=pltpu.SMEM),
        ],
        out_shape=out_shape,
        compiler_params=pltpu.CompilerParams(
            dimension_semantics=("arbitrary",),
        ),
    )(xv, enc)

    total = s[0, 0] + jnp.sum(sc_s)
    w = c[0, 0] + jnp.sum(sc_c)
    return (total / w).astype(jnp.float32)


# speedup vs baseline: 1.7915x; 1.7915x over previous
import jax
import jax.numpy as jnp
import numpy as np
from jax.experimental import pallas as pl
from jax.experimental.pallas import tpu as pltpu

_MARGINS = np.array(
    [0.45357266, 1.0, 0.49222963, 0.76696184, 1.0, 0.43823621, 0.60325897,
     0.57481898],
    dtype=np.float32,
)
_M = (2.4 * _MARGINS).astype(np.float32)
_C = 8


def _body(x_ref, enc_ref, sum_ref, cnt_ref):
    i = pl.program_id(0)

    @pl.when(i == 0)
    def _init():
        sum_ref[0, 0] = jnp.float32(0.0)
        cnt_ref[0, 0] = jnp.float32(0.0)

    Bb = enc_ref.shape[0]
    CH = 64

    acc = jnp.zeros((8, 128), jnp.float32)
    cnt = jnp.zeros((8, 128), jnp.float32)
    for k in range(Bb // CH):
        r = slice(k * CH, (k + 1) * CH)
        enc = enc_ref[r, :].astype(jnp.int32)
        tgt = enc & 7
        mskf = (enc >> 3).astype(jnp.float32)
        xt = jnp.transpose(x_ref[r, :, :], (1, 0, 2))
        S = jnp.zeros((CH, 128), jnp.float32)
        gacc = jnp.zeros((CH, 128), jnp.float32)
        for c in range(_C):
            xc = xt[c]
            sel = tgt == c
            xm = xc - jnp.float32(_M[c])
            S = S + jnp.exp(jnp.where(sel, xm, xc))
            gacc = gacc + jnp.where(sel, xm, 0.0)
        per = mskf * (jnp.log(S) - gacc)
        acc = acc + jnp.sum(per.reshape(CH // 8, 8, 128), axis=0)
        cnt = cnt + jnp.sum(mskf.reshape(CH // 8, 8, 128), axis=0)

    sum_ref[0, 0] += jnp.sum(acc)
    cnt_ref[0, 0] += jnp.sum(cnt)


@jax.jit
def kernel(x, target, mask):
    N, C = x.shape
    assert C == _C
    rows = N // 128
    xv = x.reshape(rows, 128, C).transpose(0, 2, 1)
    enc = (target | (mask.astype(jnp.int32) << 3)).astype(jnp.int8)
    enc = enc.reshape(rows, 128)

    Bb = 2048
    grid = (rows // Bb,)
    out_shape = [
        jax.ShapeDtypeStruct((1, 1), jnp.float32),
        jax.ShapeDtypeStruct((1, 1), jnp.float32),
    ]
    s, c = pl.pallas_call(
        _body,
        grid=grid,
        in_specs=[
            pl.BlockSpec((Bb, C, 128), lambda i: (i, 0, 0)),
            pl.BlockSpec((Bb, 128), lambda i: (i, 0)),
        ],
        out_specs=[
            pl.BlockSpec(memory_space=pltpu.SMEM),
            pl.BlockSpec(memory_space=pltpu.SMEM),
        ],
        out_shape=out_shape,
        compiler_params=pltpu.CompilerParams(
            dimension_semantics=("arbitrary",),
        ),
    )(xv, enc)
    return (s[0, 0] / c[0, 0]).astype(jnp.float32)
